# baseline (speedup 1.0000x reference)
import functools

import jax
import jax.numpy as jnp
import numpy as np
from jax import lax
from jax.experimental import pallas as pl
from jax.experimental.pallas import tpu as pltpu
from jax.experimental.pallas import tpu_sc as plsc

_VOCAB = 1000000
_D = 64
_SEQ = 200
_BATCH = 4096
_NC = 2
_NS = 16
_NW = _NC * _NS
_FLAT = _BATCH * _SEQ
_PER_W = _FLAT // _NW
_HALF = 100
_NBLK = _PER_W // _HALF
_SEQ_PER_W = _PER_W // _SEQ
_SCALE = 8.0


def _pos_encoding() -> np.ndarray:
    pos = np.arange(_SEQ, dtype=np.float32)[:, None]
    i = np.arange(_D, dtype=np.float32)[None, :]
    angle_rates = 1.0 / np.power(
        10000.0, (2.0 * np.floor(i / 2.0)) / np.float32(_D)
    )
    angle_rads = pos * angle_rates
    angle_rads[:, 0::2] = np.sin(angle_rads[:, 0::2])
    angle_rads[:, 1::2] = np.cos(angle_rads[:, 1::2])
    return angle_rads.astype(np.float32)


_PE = _pos_encoding()


def _emb_body(x_hbm, w_hbm, pe_hbm, out_hbm, idx_v, pe_v, rows_v, sem):
    c = lax.axis_index("c")
    s = lax.axis_index("s")
    wid = s * _NC + c
    base_row = wid * _PER_W

    pltpu.sync_copy(x_hbm.at[wid], idx_v)
    pltpu.sync_copy(pe_hbm, pe_v)

    def seq_body(sq, carry):
        g0 = pltpu.async_copy(
            w_hbm.at[idx_v.at[2 * sq]], rows_v.at[pl.ds(0, _HALF)], sem
        )
        g1 = pltpu.async_copy(
            w_hbm.at[idx_v.at[2 * sq + 1]], rows_v.at[pl.ds(_HALF, _HALF)], sem
        )
        g0.wait()
        g1.wait()

        def row_body(r, carry2):
            for l in range(_D // 16):
                sl = pl.ds(l * 16, 16)
                rows_v[r, sl] = rows_v[r, sl] * _SCALE + pe_v[r, sl]
            return carry2

        lax.fori_loop(0, _SEQ, row_body, 0, unroll=2)

        pltpu.sync_copy(
            rows_v, out_hbm.at[pl.ds(base_row + sq * _SEQ, _SEQ)]
        )
        return carry

    lax.fori_loop(0, _SEQ_PER_W, seq_body, 0)


_emb_kernel = functools.partial(
    pl.kernel,
    out_type=jax.ShapeDtypeStruct((_FLAT, _D), jnp.float32),
    mesh=plsc.VectorSubcoreMesh(core_axis_name="c", subcore_axis_name="s"),
    compiler_params=pltpu.CompilerParams(use_tc_tiling_on_sc=False),
    scratch_types=[
        pltpu.VMEM((_NBLK, _HALF), jnp.int32),
        pltpu.VMEM((_SEQ, _D), jnp.float32),
        pltpu.VMEM((_SEQ, _D), jnp.float32),
        pltpu.SemaphoreType.DMA,
    ],
)(_emb_body)


@jax.jit
def kernel(x, W):
    x_blocks = x.reshape(_NW, _NBLK, _HALF).astype(jnp.int32)
    pe = jnp.asarray(_PE)
    out = _emb_kernel(x_blocks, W, pe)
    return out.reshape(_BATCH, _SEQ, _D)


# speedup vs baseline: 1.2569x; 1.2569x over previous
import functools

import jax
import jax.numpy as jnp
import numpy as np
from jax import lax
from jax.experimental import pallas as pl
from jax.experimental.pallas import tpu as pltpu
from jax.experimental.pallas import tpu_sc as plsc

_VOCAB = 1000000
_D = 64
_SEQ = 200
_BATCH = 4096
_NC = 2
_NS = 16
_NW = _NC * _NS
_FLAT = _BATCH * _SEQ
_PER_W = _FLAT // _NW
_HALF = 100
_NBLK = _PER_W // _HALF
_SEQ_PER_W = _PER_W // _SEQ
_SCALE = 8.0


def _pos_encoding() -> np.ndarray:
    pos = np.arange(_SEQ, dtype=np.float32)[:, None]
    i = np.arange(_D, dtype=np.float32)[None, :]
    angle_rates = 1.0 / np.power(
        10000.0, (2.0 * np.floor(i / 2.0)) / np.float32(_D)
    )
    angle_rads = pos * angle_rates
    angle_rads[:, 0::2] = np.sin(angle_rads[:, 0::2])
    angle_rads[:, 1::2] = np.cos(angle_rads[:, 1::2])
    return angle_rads.astype(np.float32)


_PE = _pos_encoding()


_NBUF = 8
_DEPTH = 7


def _emb_body(x_hbm, w_hbm, pe_hbm, out_hbm, idx_v, pe_v, rows_v, gsem, ssem):
    c = lax.axis_index("c")
    s = lax.axis_index("s")
    wid = s * _NC + c
    base_row = wid * _PER_W

    pltpu.sync_copy(x_hbm.at[wid], idx_v)
    pltpu.sync_copy(pe_hbm, pe_v)

    def gather(j, b):
        return pltpu.make_async_copy(
            w_hbm.at[idx_v.at[j]], rows_v.at[b], gsem.at[b]
        )

    def store(j, b):
        return pltpu.make_async_copy(
            rows_v.at[b], out_hbm.at[pl.ds(base_row + j * _HALF, _HALF)],
            ssem.at[b],
        )

    for b in range(_DEPTH):
        gather(b, b).start()

    def outer_body(o, carry):
        for b in range(_NBUF):
            j = o * _NBUF + b
            jn = j + _DEPTH
            bg = (b + _DEPTH) % _NBUF

            def issue_next():
                def _wait_prev_store():
                    store(jn - _NBUF, bg).wait()

                if b == 0:
                    pl.when(o >= 1)(_wait_prev_store)
                else:
                    _wait_prev_store()
                gather(jn, bg).start()

            if b == 0:
                issue_next()
            else:
                pl.when(jn < _NBLK)(issue_next)

            gather(j, b).wait()
            pe_off = (b % 2) * _HALF

            def row_body(r, carry2):
                for l in range(_D // 16):
                    sl = pl.ds(l * 16, 16)
                    rows_v[b, r, sl] = (
                        rows_v[b, r, sl] * _SCALE + pe_v[pe_off + r, sl]
                    )
                return carry2

            lax.fori_loop(0, _HALF, row_body, 0, unroll=4)

            store(j, b).start()
        return carry

    lax.fori_loop(0, _NBLK // _NBUF, outer_body, 0)

    for b in range(_NBUF):
        store(_NBLK - _NBUF + b, b).wait()


_emb_kernel = functools.partial(
    pl.kernel,
    out_type=jax.ShapeDtypeStruct((_FLAT, _D), jnp.float32),
    mesh=plsc.VectorSubcoreMesh(core_axis_name="c", subcore_axis_name="s"),
    compiler_params=pltpu.CompilerParams(use_tc_tiling_on_sc=False),
    scratch_types=[
        pltpu.VMEM((_NBLK, _HALF), jnp.int32),
        pltpu.VMEM((_SEQ, _D), jnp.float32),
        pltpu.VMEM((_NBUF, _HALF, _D), jnp.float32),
        pltpu.SemaphoreType.DMA((_NBUF,)),
        pltpu.SemaphoreType.DMA((_NBUF,)),
    ],
)(_emb_body)


@jax.jit
def kernel(x, W):
    x_blocks = x.reshape(_NW, _NBLK, _HALF).astype(jnp.int32)
    pe = jnp.asarray(_PE)
    out = _emb_kernel(x_blocks, W, pe)
    return out.reshape(_BATCH, _SEQ, _D)


# speedup vs baseline: 1.2623x; 1.0043x over previous
import functools

import jax
import jax.numpy as jnp
import numpy as np
from jax import lax
from jax.experimental import pallas as pl
from jax.experimental.pallas import tpu as pltpu
from jax.experimental.pallas import tpu_sc as plsc

_VOCAB = 1000000
_D = 64
_SEQ = 200
_BATCH = 4096
_NC = 2
_NS = 16
_NW = _NC * _NS
_FLAT = _BATCH * _SEQ
_PER_W = _FLAT // _NW
_HALF = 100
_NBLK = _PER_W // _HALF
_SEQ_PER_W = _PER_W // _SEQ
_SCALE = 8.0


def _pos_encoding() -> np.ndarray:
    pos = np.arange(_SEQ, dtype=np.float32)[:, None]
    i = np.arange(_D, dtype=np.float32)[None, :]
    angle_rates = 1.0 / np.power(
        10000.0, (2.0 * np.floor(i / 2.0)) / np.float32(_D)
    )
    angle_rads = pos * angle_rates
    angle_rads[:, 0::2] = np.sin(angle_rads[:, 0::2])
    angle_rads[:, 1::2] = np.cos(angle_rads[:, 1::2])
    return angle_rads.astype(np.float32)


_PE = _pos_encoding()


_NBUF = 8
_DEPTH = 7


def _emb_body(x_hbm, w_hbm, pe_hbm, out_hbm, idx_v, pe_v, rows_v, gsem, ssem):
    c = lax.axis_index("c")
    s = lax.axis_index("s")
    wid = s * _NC + c
    base_row = wid * _PER_W

    pltpu.sync_copy(x_hbm.at[wid], idx_v)
    pltpu.sync_copy(pe_hbm, pe_v)

    def gather(j, b):
        return pltpu.make_async_copy(
            w_hbm.at[idx_v.at[j]], rows_v.at[b], gsem.at[b]
        )

    def store(o, b):
        bi = wid * (_PER_W // _SEQ) + o * (_NBUF // 2) + b // 2
        return pltpu.make_async_copy(
            rows_v.at[b],
            out_hbm.at[bi, pl.ds((b % 2) * _HALF, _HALF)],
            ssem.at[b],
        )

    for b in range(_DEPTH):
        gather(b, b).start()

    def outer_body(o, carry):
        for b in range(_NBUF):
            j = o * _NBUF + b
            jn = j + _DEPTH
            bg = (b + _DEPTH) % _NBUF

            def issue_next():
                def _wait_prev_store():
                    if b == 0:
                        store(o - 1, 7).wait()
                    else:
                        store(o, b - 1).wait()

                if b == 0:
                    pl.when(o >= 1)(_wait_prev_store)
                else:
                    _wait_prev_store()
                gather(jn, bg).start()

            if b == 0:
                issue_next()
            else:
                pl.when(jn < _NBLK)(issue_next)

            gather(j, b).wait()
            pe_off = (b % 2) * _HALF

            def row_body(r, carry2):
                for l in range(_D // 16):
                    sl = pl.ds(l * 16, 16)
                    rows_v[b, r, sl] = (
                        rows_v[b, r, sl] * _SCALE + pe_v[pe_off + r, sl]
                    )
                return carry2

            lax.fori_loop(0, _HALF, row_body, 0, unroll=4)

            store(o, b).start()
        return carry

    lax.fori_loop(0, _NBLK // _NBUF, outer_body, 0)

    for b in range(_NBUF):
        store(_NBLK // _NBUF - 1, b).wait()


_emb_kernel = functools.partial(
    pl.kernel,
    out_type=jax.ShapeDtypeStruct((_BATCH, _SEQ, _D), jnp.float32),
    mesh=plsc.VectorSubcoreMesh(core_axis_name="c", subcore_axis_name="s"),
    compiler_params=pltpu.CompilerParams(use_tc_tiling_on_sc=False),
    scratch_types=[
        pltpu.VMEM((_NBLK, _HALF), jnp.int32),
        pltpu.VMEM((_SEQ, _D), jnp.float32),
        pltpu.VMEM((_NBUF, _HALF, _D), jnp.float32),
        pltpu.SemaphoreType.DMA((_NBUF,)),
        pltpu.SemaphoreType.DMA((_NBUF,)),
    ],
)(_emb_body)


@jax.jit
def kernel(x, W):
    x_blocks = x.reshape(_NW, _NBLK, _HALF).astype(jnp.int32)
    pe = jnp.asarray(_PE)
    return _emb_kernel(x_blocks, W, pe)
